# baseline (speedup 1.0000x reference)
import functools

import jax
import jax.numpy as jnp
from jax import lax
from jax.experimental import pallas as pl
from jax.experimental.pallas import tpu as pltpu
from jax.experimental.pallas import tpu_sc as plsc

N = 10000
E = 320000
C = 128
CA = 144
NC = 2
NS = 16
NW = NC * NS
EPW = E // NW
CH = 125
NCHUNK = EPW // CH
NQ = 5
CPQ = NCHUNK // NQ
WCH = 80
NWB = N // WCH
ITS = (NWB + NS - 1) // NS

_SC_PARAMS = pltpu.CompilerParams(use_tc_tiling_on_sc=False)
_MESH = dict(core_axis_name="c", subcore_axis_name="s")


def _sc_aggregate(x_aug, src_r, dst_r, zeros):
    @functools.partial(
        pl.kernel,
        out_type=jax.ShapeDtypeStruct((NC, N, CA), jnp.float32),
        mesh=plsc.VectorSubcoreMesh(**_MESH),
        compiler_params=_SC_PARAMS,
        scratch_types=[
            pltpu.VMEM((CPQ, CH), jnp.int32),
            pltpu.VMEM((CPQ, CH), jnp.int32),
            pltpu.VMEM((2, CH, CA), jnp.float32),
            pltpu.VMEM_SHARED((N, CA), jnp.float32),
            pltpu.SemaphoreType.DMA,
        ],
    )
    def agg(x_hbm, src_hbm, dst_hbm, z_hbm, out_hbm,
            src_v, dst_v, rows_v, acc_sh, sem):
        cid = lax.axis_index("c")
        sid = lax.axis_index("s")
        wid = cid * NS + sid

        pltpu.sync_copy(z_hbm, rows_v.at[0])

        for it in range(ITS):
            cidx = it * NS + sid

            @pl.when(cidx < NWB)
            def _zero_chunk():
                pltpu.sync_copy(rows_v.at[0, pl.ds(0, WCH)],
                                acc_sh.at[pl.ds(cidx * WCH, WCH)])

        plsc.subcore_barrier()

        def fire(j, pb):
            pltpu.async_copy(x_hbm.at[src_v.at[j]], rows_v.at[pb], sem)

        def drain(j, pb):
            pltpu.make_async_copy(
                x_hbm.at[src_v.at[j]], rows_v.at[pb], sem).wait()

        def scatter(j, pb):
            pltpu.sync_copy(rows_v.at[pb],
                            acc_sh.at[dst_v.at[j]], add=True)

        for q in range(NQ):
            pltpu.sync_copy(src_hbm.at[wid, q], src_v)
            pltpu.sync_copy(dst_hbm.at[wid, q], dst_v)
            fire(0, 0)

            @pl.loop(0, CPQ // 2)
            def _grp(h):
                j0 = 2 * h
                drain(j0, 0)
                fire(j0 + 1, 1)
                scatter(j0, 0)
                j1 = j0 + 1
                drain(j1, 1)

                @pl.when(j1 + 1 < CPQ)
                def _refill():
                    fire(j1 + 1, 0)

                scatter(j1, 1)

        plsc.subcore_barrier()

        for it in range(ITS):
            cidx = it * NS + sid

            @pl.when(cidx < NWB)
            def _write_chunk():
                r0 = cidx * WCH
                pltpu.sync_copy(acc_sh.at[pl.ds(r0, WCH)],
                                rows_v.at[0, pl.ds(0, WCH)])
                pltpu.sync_copy(rows_v.at[0, pl.ds(0, WCH)],
                                out_hbm.at[cid, pl.ds(r0, WCH)])

    return agg(x_aug, src_r, dst_r, zeros)


BR = 2000


def _tc_body(p_ref, x_ref, wlT_ref, wrT_ref, b_ref, o_ref):
    s = p_ref[0, :, :C] + p_ref[1, :, :C]
    d = p_ref[0, :, C:C + 1] + p_ref[1, :, C:C + 1]
    m = s / jnp.maximum(d, 1.0)
    o_ref[...] = (
        jnp.dot(m, wlT_ref[...], preferred_element_type=jnp.float32,
                precision=lax.Precision.HIGHEST)
        + jnp.dot(x_ref[...], wrT_ref[...], preferred_element_type=jnp.float32,
                  precision=lax.Precision.HIGHEST)
        + b_ref[...]
    )


def _tc_combine(p, x, wlT, wrT, b):
    return pl.pallas_call(
        _tc_body,
        grid=(N // BR,),
        in_specs=[
            pl.BlockSpec((NC, BR, CA), lambda i: (0, i, 0)),
            pl.BlockSpec((BR, C), lambda i: (i, 0)),
            pl.BlockSpec((C, C), lambda i: (0, 0)),
            pl.BlockSpec((C, C), lambda i: (0, 0)),
            pl.BlockSpec((1, C), lambda i: (0, 0)),
        ],
        out_specs=pl.BlockSpec((BR, C), lambda i: (i, 0)),
        out_shape=jax.ShapeDtypeStruct((N, C), jnp.float32),
    )(p, x, wlT, wrT, b)


def kernel(x, edge_index, W_l, b_l, W_r):
    src = edge_index[0].reshape(NW, NQ, CPQ, CH)
    dst = edge_index[1].reshape(NW, NQ, CPQ, CH)
    x_aug = jnp.concatenate(
        [x, jnp.ones((N, 1), jnp.float32), jnp.zeros((N, CA - C - 1), jnp.float32)],
        axis=1)
    zeros = jnp.zeros((CH, CA), jnp.float32)
    parts = _sc_aggregate(x_aug, src, dst, zeros)
    return _tc_combine(parts, x, W_l.T, W_r.T, b_l.reshape(1, C))


# speedup vs baseline: 1.2000x; 1.2000x over previous
import functools

import jax
import jax.numpy as jnp
from jax import lax
from jax.experimental import pallas as pl
from jax.experimental.pallas import tpu as pltpu
from jax.experimental.pallas import tpu_sc as plsc

N = 10000
E = 320000
C = 128
NC = 2
NS = 16
NW = NC * NS
EPW = E // NW
CH = 125
NCHUNK = EPW // CH
NQ = 5
CPQ = NCHUNK // NQ
DW = 16
WCH = 80
NWB = N // WCH
ITS = (NWB + NS - 1) // NS
AUXZ = 128

_SC_PARAMS = pltpu.CompilerParams(use_tc_tiling_on_sc=False)
_MESH = dict(core_axis_name="c", subcore_axis_name="s")


def _sc_aggregate(x, src_r, dst_r, zeros, aux):
    @functools.partial(
        pl.kernel,
        out_type=[
            jax.ShapeDtypeStruct((NC, N, C), jnp.float32),
            jax.ShapeDtypeStruct((NC, N, DW), jnp.float32),
        ],
        mesh=plsc.VectorSubcoreMesh(**_MESH),
        compiler_params=_SC_PARAMS,
        scratch_types=[
            pltpu.VMEM((CPQ, CH), jnp.int32),
            pltpu.VMEM((CPQ, CH), jnp.int32),
            pltpu.VMEM((2, CH, C), jnp.float32),
            pltpu.VMEM((CH, DW), jnp.float32),
            pltpu.VMEM((WCH, DW), jnp.float32),
            pltpu.VMEM_SHARED((N, C), jnp.float32),
            pltpu.VMEM_SHARED((N, DW), jnp.float32),
            pltpu.SemaphoreType.DMA,
            pltpu.SemaphoreType.DMA,
        ],
    )
    def agg(x_hbm, src_hbm, dst_hbm, z_hbm, aux_hbm, out_hbm, deg_hbm,
            src_v, dst_v, rows_v, ones_v, dbuf, acc_sh, deg_sh, sem, dsem):
        cid = lax.axis_index("c")
        sid = lax.axis_index("s")
        wid = cid * NS + sid

        pltpu.sync_copy(z_hbm, rows_v.at[0])
        pltpu.sync_copy(aux_hbm.at[pl.ds(0, CH)], ones_v)
        pltpu.sync_copy(aux_hbm.at[pl.ds(AUXZ, WCH)], dbuf)

        for it in range(ITS):
            cidx = it * NS + sid

            @pl.when(cidx < NWB)
            def _zero_chunk():
                pltpu.sync_copy(rows_v.at[0, pl.ds(0, WCH)],
                                acc_sh.at[pl.ds(cidx * WCH, WCH)])
                pltpu.sync_copy(dbuf, deg_sh.at[pl.ds(cidx * WCH, WCH)])

        plsc.subcore_barrier()

        def fire(j, pb):
            pltpu.async_copy(x_hbm.at[src_v.at[j]], rows_v.at[pb], sem)

        def drain(j, pb):
            pltpu.make_async_copy(
                x_hbm.at[src_v.at[j]], rows_v.at[pb], sem).wait()

        def scatter(j, pb):
            pltpu.sync_copy(rows_v.at[pb],
                            acc_sh.at[dst_v.at[j]], add=True)
            pltpu.async_copy(ones_v, deg_sh.at[dst_v.at[j]], dsem, add=True)

        for q in range(NQ):
            pltpu.sync_copy(src_hbm.at[wid, q], src_v)
            pltpu.sync_copy(dst_hbm.at[wid, q], dst_v)
            fire(0, 0)

            @pl.loop(0, CPQ // 2)
            def _grp(h):
                j0 = 2 * h
                drain(j0, 0)
                fire(j0 + 1, 1)
                scatter(j0, 0)
                j1 = j0 + 1
                drain(j1, 1)

                @pl.when(j1 + 1 < CPQ)
                def _refill():
                    fire(j1 + 1, 0)

                scatter(j1, 1)

            for _ in range(CPQ):
                pltpu.make_async_copy(
                    ones_v, deg_sh.at[dst_v.at[0]], dsem).wait()

        plsc.subcore_barrier()

        for it in range(ITS):
            cidx = it * NS + sid

            @pl.when(cidx < NWB)
            def _write_chunk():
                r0 = cidx * WCH
                pltpu.sync_copy(acc_sh.at[pl.ds(r0, WCH)],
                                rows_v.at[0, pl.ds(0, WCH)])
                pltpu.sync_copy(rows_v.at[0, pl.ds(0, WCH)],
                                out_hbm.at[cid, pl.ds(r0, WCH)])
                pltpu.sync_copy(deg_sh.at[pl.ds(r0, WCH)], dbuf)
                pltpu.sync_copy(dbuf, deg_hbm.at[cid, pl.ds(r0, WCH)])

    return agg(x, src_r, dst_r, zeros, aux)


BR = 2000


def _tc_body(p_ref, deg_ref, x_ref, wlT_ref, wrT_ref, b_ref, o_ref):
    s = p_ref[0] + p_ref[1]
    d = deg_ref[0, :, 0:1] + deg_ref[1, :, 0:1]
    m = s / jnp.maximum(d, 1.0)
    o_ref[...] = (
        jnp.dot(m, wlT_ref[...], preferred_element_type=jnp.float32,
                precision=lax.Precision.HIGHEST)
        + jnp.dot(x_ref[...], wrT_ref[...], preferred_element_type=jnp.float32,
                  precision=lax.Precision.HIGHEST)
        + b_ref[...]
    )


def _tc_combine(p, degp, x, wlT, wrT, b):
    return pl.pallas_call(
        _tc_body,
        grid=(N // BR,),
        in_specs=[
            pl.BlockSpec((NC, BR, C), lambda i: (0, i, 0)),
            pl.BlockSpec((NC, BR, DW), lambda i: (0, i, 0)),
            pl.BlockSpec((BR, C), lambda i: (i, 0)),
            pl.BlockSpec((C, C), lambda i: (0, 0)),
            pl.BlockSpec((C, C), lambda i: (0, 0)),
            pl.BlockSpec((1, C), lambda i: (0, 0)),
        ],
        out_specs=pl.BlockSpec((BR, C), lambda i: (i, 0)),
        out_shape=jax.ShapeDtypeStruct((N, C), jnp.float32),
    )(p, degp, x, wlT, wrT, b)


def kernel(x, edge_index, W_l, b_l, W_r):
    src = edge_index[0].reshape(NW, NQ, CPQ, CH)
    dst = edge_index[1].reshape(NW, NQ, CPQ, CH)
    zeros = jnp.zeros((CH, C), jnp.float32)
    aux = jnp.concatenate(
        [jnp.ones((CH, DW), jnp.float32),
         jnp.zeros((AUXZ - CH + WCH, DW), jnp.float32)])
    parts, degs = _sc_aggregate(x, src, dst, zeros, aux)
    return _tc_combine(parts, degs, x, W_l.T, W_r.T, b_l.reshape(1, C))


# speedup vs baseline: 1.2003x; 1.0002x over previous
import functools

import jax
import jax.numpy as jnp
from jax import lax
from jax.experimental import pallas as pl
from jax.experimental.pallas import tpu as pltpu
from jax.experimental.pallas import tpu_sc as plsc

N = 10000
E = 320000
C = 128
NC = 2
NS = 16
NW = NC * NS
EPW = E // NW
CH = 125
NCHUNK = EPW // CH
NQ = 5
CPQ = NCHUNK // NQ
DW = 16
WCH = 80
NWB = N // WCH
ITS = (NWB + NS - 1) // NS
AUXZ = 128

_SC_PARAMS = pltpu.CompilerParams(use_tc_tiling_on_sc=False)
_MESH = dict(core_axis_name="c", subcore_axis_name="s")


def _sc_aggregate(x, src_r, dst_r, zeros, aux):
    @functools.partial(
        pl.kernel,
        out_type=[
            jax.ShapeDtypeStruct((NC, N, C), jnp.float32),
            jax.ShapeDtypeStruct((NC, N, DW), jnp.float32),
        ],
        mesh=plsc.VectorSubcoreMesh(**_MESH),
        compiler_params=_SC_PARAMS,
        scratch_types=[
            pltpu.VMEM((CPQ, CH), jnp.int32),
            pltpu.VMEM((CPQ, CH), jnp.int32),
            pltpu.VMEM((2, CH, C), jnp.float32),
            pltpu.VMEM((CH, DW), jnp.float32),
            pltpu.VMEM((WCH, DW), jnp.float32),
            pltpu.VMEM_SHARED((N, C), jnp.float32),
            pltpu.VMEM_SHARED((N, DW), jnp.float32),
            pltpu.SemaphoreType.DMA,
            pltpu.SemaphoreType.DMA,
            pltpu.SemaphoreType.DMA,
            pltpu.SemaphoreType.DMA,
        ],
    )
    def agg(x_hbm, src_hbm, dst_hbm, z_hbm, aux_hbm, out_hbm, deg_hbm,
            src_v, dst_v, rows_v, ones_v, dbuf, acc_sh, deg_sh,
            sem, dsem, ssem0, ssem1):
        cid = lax.axis_index("c")
        sid = lax.axis_index("s")
        wid = cid * NS + sid

        pltpu.sync_copy(z_hbm, rows_v.at[0])
        pltpu.sync_copy(aux_hbm.at[pl.ds(0, CH)], ones_v)
        pltpu.sync_copy(aux_hbm.at[pl.ds(AUXZ, WCH)], dbuf)

        for it in range(ITS):
            cidx = it * NS + sid

            @pl.when(cidx < NWB)
            def _zero_chunk():
                pltpu.sync_copy(rows_v.at[0, pl.ds(0, WCH)],
                                acc_sh.at[pl.ds(cidx * WCH, WCH)])
                pltpu.sync_copy(dbuf, deg_sh.at[pl.ds(cidx * WCH, WCH)])

        plsc.subcore_barrier()

        def fire(j, pb):
            pltpu.async_copy(x_hbm.at[src_v.at[j]], rows_v.at[pb], sem)

        def drain(j, pb):
            pltpu.make_async_copy(
                x_hbm.at[src_v.at[j]], rows_v.at[pb], sem).wait()

        ssem = (ssem0, ssem1)

        def fire_s(j, pb):
            pltpu.async_copy(rows_v.at[pb], acc_sh.at[dst_v.at[j]],
                             ssem[pb], add=True)
            pltpu.async_copy(ones_v, deg_sh.at[dst_v.at[j]], dsem, add=True)

        def wait_s(j, pb):
            pltpu.make_async_copy(rows_v.at[pb], acc_sh.at[dst_v.at[j]],
                                  ssem[pb]).wait()

        for q in range(NQ):
            pltpu.sync_copy(src_hbm.at[wid, q], src_v)
            pltpu.sync_copy(dst_hbm.at[wid, q], dst_v)
            fire(0, 0)

            @pl.loop(0, CPQ // 2)
            def _grp(h):
                j0 = 2 * h
                drain(j0, 0)
                fire_s(j0, 0)

                @pl.when(j0 > 0)
                def _wait_prev():
                    wait_s(j0 - 1, 1)

                fire(j0 + 1, 1)
                j1 = j0 + 1
                drain(j1, 1)
                fire_s(j1, 1)
                wait_s(j0, 0)

                @pl.when(j1 + 1 < CPQ)
                def _refill():
                    fire(j1 + 1, 0)

            wait_s(CPQ - 1, 1)

            for _ in range(CPQ):
                pltpu.make_async_copy(
                    ones_v, deg_sh.at[dst_v.at[0]], dsem).wait()

        plsc.subcore_barrier()

        for it in range(ITS):
            cidx = it * NS + sid

            @pl.when(cidx < NWB)
            def _write_chunk():
                r0 = cidx * WCH
                pltpu.sync_copy(acc_sh.at[pl.ds(r0, WCH)],
                                rows_v.at[0, pl.ds(0, WCH)])
                pltpu.sync_copy(rows_v.at[0, pl.ds(0, WCH)],
                                out_hbm.at[cid, pl.ds(r0, WCH)])
                pltpu.sync_copy(deg_sh.at[pl.ds(r0, WCH)], dbuf)
                pltpu.sync_copy(dbuf, deg_hbm.at[cid, pl.ds(r0, WCH)])

    return agg(x, src_r, dst_r, zeros, aux)


BR = 2000


def _tc_body(p_ref, deg_ref, x_ref, wlT_ref, wrT_ref, b_ref, o_ref):
    s = p_ref[0] + p_ref[1]
    d = deg_ref[0, :, 0:1] + deg_ref[1, :, 0:1]
    m = s / jnp.maximum(d, 1.0)
    o_ref[...] = (
        jnp.dot(m, wlT_ref[...], preferred_element_type=jnp.float32,
                precision=lax.Precision.HIGHEST)
        + jnp.dot(x_ref[...], wrT_ref[...], preferred_element_type=jnp.float32,
                  precision=lax.Precision.HIGHEST)
        + b_ref[...]
    )


def _tc_combine(p, degp, x, wlT, wrT, b):
    return pl.pallas_call(
        _tc_body,
        grid=(N // BR,),
        in_specs=[
            pl.BlockSpec((NC, BR, C), lambda i: (0, i, 0)),
            pl.BlockSpec((NC, BR, DW), lambda i: (0, i, 0)),
            pl.BlockSpec((BR, C), lambda i: (i, 0)),
            pl.BlockSpec((C, C), lambda i: (0, 0)),
            pl.BlockSpec((C, C), lambda i: (0, 0)),
            pl.BlockSpec((1, C), lambda i: (0, 0)),
        ],
        out_specs=pl.BlockSpec((BR, C), lambda i: (i, 0)),
        out_shape=jax.ShapeDtypeStruct((N, C), jnp.float32),
    )(p, degp, x, wlT, wrT, b)


def kernel(x, edge_index, W_l, b_l, W_r):
    src = edge_index[0].reshape(NW, NQ, CPQ, CH)
    dst = edge_index[1].reshape(NW, NQ, CPQ, CH)
    zeros = jnp.zeros((CH, C), jnp.float32)
    aux = jnp.concatenate(
        [jnp.ones((CH, DW), jnp.float32),
         jnp.zeros((AUXZ - CH + WCH, DW), jnp.float32)])
    parts, degs = _sc_aggregate(x, src, dst, zeros, aux)
    return _tc_combine(parts, degs, x, W_l.T, W_r.T, b_l.reshape(1, C))


# speedup vs baseline: 1.3149x; 1.0955x over previous
import functools

import jax
import jax.numpy as jnp
from jax import lax
from jax.experimental import pallas as pl
from jax.experimental.pallas import tpu as pltpu
from jax.experimental.pallas import tpu_sc as plsc

N = 10000
E = 320000
C = 128
NC = 2
NS = 16
NW = NC * NS
EPW = E // NW
CH = 125
NCHUNK = EPW // CH
NQ = 5
CPQ = NCHUNK // NQ
DW = 16
WCH = 80
NWB = N // WCH
ITS = (NWB + NS - 1) // NS
AUXZ = 128

_SC_PARAMS = pltpu.CompilerParams(use_tc_tiling_on_sc=False)
_MESH = dict(core_axis_name="c", subcore_axis_name="s")


def _sc_aggregate(x, eidx):
    @functools.partial(
        pl.kernel,
        out_type=[
            jax.ShapeDtypeStruct((NC, N, C), jnp.float32),
            jax.ShapeDtypeStruct((NC, N, DW), jnp.float32),
        ],
        mesh=plsc.VectorSubcoreMesh(**_MESH),
        compiler_params=_SC_PARAMS,
        scratch_types=[
            pltpu.VMEM((CPQ, CH), jnp.int32),
            pltpu.VMEM((CPQ, CH), jnp.int32),
            pltpu.VMEM((2, CH, C), jnp.float32),
            pltpu.VMEM((CH, DW), jnp.float32),
            pltpu.VMEM((WCH, DW), jnp.float32),
            pltpu.VMEM_SHARED((N, C), jnp.float32),
            pltpu.VMEM_SHARED((N, DW), jnp.float32),
            pltpu.SemaphoreType.DMA,
            pltpu.SemaphoreType.DMA,
            pltpu.SemaphoreType.DMA,
            pltpu.SemaphoreType.DMA,
        ],
    )
    def agg(x_hbm, e_hbm, out_hbm, deg_hbm,
            src_v, dst_v, rows_v, ones_v, dbuf, acc_sh, deg_sh,
            sem, dsem, ssem0, ssem1):
        cid = lax.axis_index("c")
        sid = lax.axis_index("s")
        wid = cid * NS + sid

        zrow = jnp.zeros((16,), jnp.float32)
        orow = jnp.ones((16,), jnp.float32)

        @pl.loop(0, WCH)
        def _fill_consts(r):
            for cc in range(C // 16):
                rows_v[0, r, pl.ds(cc * 16, 16)] = zrow
            ones_v[r, :] = orow
            dbuf[r, :] = zrow

        @pl.loop(WCH, CH)
        def _fill_tail(r):
            ones_v[r, :] = orow

        for it in range(ITS):
            cidx = it * NS + sid

            @pl.when(cidx < NWB)
            def _zero_chunk():
                pltpu.sync_copy(rows_v.at[0, pl.ds(0, WCH)],
                                acc_sh.at[pl.ds(cidx * WCH, WCH)])
                pltpu.sync_copy(dbuf, deg_sh.at[pl.ds(cidx * WCH, WCH)])

        plsc.subcore_barrier()

        def fire(j, pb):
            pltpu.async_copy(x_hbm.at[src_v.at[j]], rows_v.at[pb], sem)

        def drain(j, pb):
            pltpu.make_async_copy(
                x_hbm.at[src_v.at[j]], rows_v.at[pb], sem).wait()

        ssem = (ssem0, ssem1)

        def fire_s(j, pb):
            pltpu.async_copy(rows_v.at[pb], acc_sh.at[dst_v.at[j]],
                             ssem[pb], add=True)
            pltpu.async_copy(ones_v, deg_sh.at[dst_v.at[j]], dsem, add=True)

        def wait_s(j, pb):
            pltpu.make_async_copy(rows_v.at[pb], acc_sh.at[dst_v.at[j]],
                                  ssem[pb]).wait()

        for q in range(NQ):
            pltpu.sync_copy(e_hbm.at[0, wid, q], src_v)
            pltpu.sync_copy(e_hbm.at[1, wid, q], dst_v)
            fire(0, 0)

            @pl.loop(0, CPQ // 2)
            def _grp(h):
                j0 = 2 * h
                drain(j0, 0)
                fire_s(j0, 0)

                @pl.when(j0 > 0)
                def _wait_prev():
                    wait_s(j0 - 1, 1)

                fire(j0 + 1, 1)
                j1 = j0 + 1
                drain(j1, 1)
                fire_s(j1, 1)
                wait_s(j0, 0)

                @pl.when(j1 + 1 < CPQ)
                def _refill():
                    fire(j1 + 1, 0)

            wait_s(CPQ - 1, 1)

            for _ in range(CPQ):
                pltpu.make_async_copy(
                    ones_v, deg_sh.at[dst_v.at[0]], dsem).wait()

        plsc.subcore_barrier()

        for it in range(ITS):
            cidx = it * NS + sid

            @pl.when(cidx < NWB)
            def _write_chunk():
                r0 = cidx * WCH
                pltpu.sync_copy(acc_sh.at[pl.ds(r0, WCH)],
                                rows_v.at[0, pl.ds(0, WCH)])
                pltpu.sync_copy(rows_v.at[0, pl.ds(0, WCH)],
                                out_hbm.at[cid, pl.ds(r0, WCH)])
                pltpu.sync_copy(deg_sh.at[pl.ds(r0, WCH)], dbuf)
                pltpu.sync_copy(dbuf, deg_hbm.at[cid, pl.ds(r0, WCH)])

    return agg(x, eidx)


BR = 2000


def _tc_body(p_ref, deg_ref, x_ref, wlT_ref, wrT_ref, b_ref, o_ref):
    s = p_ref[0] + p_ref[1]
    d = deg_ref[0, :, 0:1] + deg_ref[1, :, 0:1]
    m = s / jnp.maximum(d, 1.0)
    o_ref[...] = (
        jnp.dot(m, wlT_ref[...], preferred_element_type=jnp.float32,
                precision=lax.Precision.HIGHEST)
        + jnp.dot(x_ref[...], wrT_ref[...], preferred_element_type=jnp.float32,
                  precision=lax.Precision.HIGHEST)
        + b_ref[...]
    )


def _tc_combine(p, degp, x, wlT, wrT, b):
    return pl.pallas_call(
        _tc_body,
        grid=(N // BR,),
        in_specs=[
            pl.BlockSpec((NC, BR, C), lambda i: (0, i, 0)),
            pl.BlockSpec((NC, BR, DW), lambda i: (0, i, 0)),
            pl.BlockSpec((BR, C), lambda i: (i, 0)),
            pl.BlockSpec((C, C), lambda i: (0, 0)),
            pl.BlockSpec((C, C), lambda i: (0, 0)),
            pl.BlockSpec((1, C), lambda i: (0, 0)),
        ],
        out_specs=pl.BlockSpec((BR, C), lambda i: (i, 0)),
        out_shape=jax.ShapeDtypeStruct((N, C), jnp.float32),
    )(p, degp, x, wlT, wrT, b)


def kernel(x, edge_index, W_l, b_l, W_r):
    eidx = edge_index.reshape(2, NW, NQ, CPQ, CH)
    parts, degs = _sc_aggregate(x, eidx)
    return _tc_combine(parts, degs, x, W_l.T, W_r.T, b_l.reshape(1, C))
